# baseline (speedup 1.0000x reference)
import functools

import jax
import jax.numpy as jnp
from jax import lax
from jax.experimental import pallas as pl
from jax.experimental.pallas import tpu as pltpu
from jax.experimental.pallas import tpu_sc as plsc

N = 10000
NP = 10240
K = 16
BIGF = 3.0e38
BIGI = 2 ** 30
PADQ = BIGF / 16
NSUB = 32
ROWS_PER_SUB = NP // NSUB
CHUNK = 8
NCHUNK = ROWS_PER_SUB // CHUNK
RB = 128


def _mlp_body(x_ref, w1, b1, w2, b2, w3, b3, ws, bs, wh, bh,
              d_ref, s_ref, h_ref):
    x = x_ref[...]
    t = jnp.tanh(jnp.dot(x, w1[...]) + b1[...])
    t = jnp.tanh(jnp.dot(t, w2[...]) + b2[...])
    d = jnp.dot(t, w3[...]) + b3[...]
    s = jnp.dot(d, ws[...]) + bs[...]
    s_ref[...] = s
    d_ref[...] = d
    h_ref[...] = jnp.dot(d, wh[...]) + bh[...]


def _mlp(x, w1, b1, w2, b2, w3, b3, ws, bs, wh, bh):
    in_dim = x.shape[1]
    full = lambda shape: pl.BlockSpec(shape, lambda r: (0, 0))
    return pl.pallas_call(
        _mlp_body,
        grid=(NP // 256,),
        in_specs=[
            pl.BlockSpec((256, in_dim), lambda r: (r, 0)),
            full((in_dim, 64)), full((1, 64)),
            full((64, 64)), full((1, 64)),
            full((64, 64)), full((1, 64)),
            full((64, 4)), full((1, 4)),
            full((64, 16)), full((1, 16)),
        ],
        out_specs=[
            pl.BlockSpec((256, 64), lambda r: (r, 0)),
            pl.BlockSpec((256, 4), lambda r: (r, 0)),
            pl.BlockSpec((256, 16), lambda r: (r, 0)),
        ],
        out_shape=[
            jax.ShapeDtypeStruct((NP, 64), jnp.float32),
            jax.ShapeDtypeStruct((NP, 4), jnp.float32),
            jax.ShapeDtypeStruct((NP, 16), jnp.float32),
        ],
    )(x, w1, b1, w2, b2, w3, b3, ws, bs, wh, bh)


NCELL = 640
FOLD = NP // NCELL


def _dist_body(s_rows, s_all, sq_rows, sq_row_t, d2_ref, fold_ref):
    t = lax.dot_general(s_rows[...], s_all[...],
                        (((1,), (1,)), ((), ())))
    d2 = (sq_rows[...] + sq_row_t[...]) - 2.0 * t
    d2_ref[...] = d2
    f = d2[:, 0:NCELL]
    for m in range(1, FOLD):
        f = jnp.minimum(f, d2[:, m * NCELL:(m + 1) * NCELL])
    fold_ref[...] = f


def _dist(s, sq, sq_t):
    return pl.pallas_call(
        _dist_body,
        grid=(NP // RB,),
        in_specs=[
            pl.BlockSpec((RB, 4), lambda r: (r, 0)),
            pl.BlockSpec((NP, 4), lambda r: (0, 0)),
            pl.BlockSpec((RB, 1), lambda r: (r, 0)),
            pl.BlockSpec((1, NP), lambda r: (0, 0)),
        ],
        out_specs=[
            pl.BlockSpec((RB, NP), lambda r: (r, 0)),
            pl.BlockSpec((RB, NCELL), lambda r: (r, 0)),
        ],
        out_shape=[
            jax.ShapeDtypeStruct((NP, NP), jnp.float32),
            jax.ShapeDtypeStruct((NP, NCELL), jnp.float32),
        ],
    )(s, s, sq, sq_t)


def _scagg_body(d2f_hbm, s4f_hbm, h_hbm,
                mean_hbm, max_hbm, idx_hbm, w_hbm,
                buf, s4v, idxsel_v, w_v, h_v, mean_f, max_f, sem):
    cid = lax.axis_index("c")
    sid = lax.axis_index("s")
    wid = sid * 2 + cid
    base = wid * ROWS_PER_SUB
    pltpu.sync_copy(s4f_hbm, s4v)
    iota16 = lax.iota(jnp.int32, 16)

    def insert_group(v, vidx, st):
        bv, bi, tau, ei = st

        def w_cond(c):
            bv, bi, v, tau, ei = c
            return jnp.any((v < tau) | ((v == tau) & (vidx < ei)))

        def w_body(c):
            bv, bi, v, tau, ei = c
            mn = jnp.min(v)
            ci = jnp.min(jnp.where(v == mn, vidx, BIGI))
            qm = bi == ei
            bv = jnp.where(qm, mn, bv)
            bi = jnp.where(qm, ci, bi)
            v = jnp.where(vidx == ci, BIGF, v)
            tau = jnp.max(bv)
            ei = jnp.max(jnp.where(bv == tau, bi, -BIGI))
            return bv, bi, v, tau, ei

        bv, bi, _, tau, ei = lax.while_loop(
            w_cond, w_body, (bv, bi, v, tau, ei))
        return bv, bi, tau, ei

    def chunk_body(ci_, _):
        row0 = base + ci_ * CHUNK
        pltpu.sync_copy(d2f_hbm.at[pl.ds(row0 * NP, CHUNK * NP)], buf)

        def row_scan(r, _):
            row = row0 + r

            def g4_body(i, st):
                off = r * NP + i * 64
                v0 = buf[pl.ds(off, 16)]
                v1 = buf[pl.ds(off + 16, 16)]
                v2 = buf[pl.ds(off + 32, 16)]
                v3 = buf[pl.ds(off + 48, 16)]
                tau = st[2]
                hit = jnp.any((v0 <= tau) | (v1 <= tau)
                              | (v2 <= tau) | (v3 <= tau))

                def do_insert(st):
                    for j, v in enumerate((v0, v1, v2, v3)):
                        st = insert_group(v, iota16 + (i * 64 + j * 16), st)
                    return st

                return lax.cond(hit, do_insert, lambda st: st, st)

            st0 = (jnp.full((16,), BIGF, jnp.float32), iota16 - 16,
                   jnp.float32(BIGF), jnp.int32(-1))
            bv, bi, _, _ = lax.fori_loop(0, NP // 64, g4_body, st0)

            d2e = jnp.zeros((16,), jnp.float32)
            for dim in range(4):
                sj = plsc.load_gather(s4v, [bi * 4 + dim])
                si = plsc.load_gather(
                    s4v, [jnp.full((16,), row * 4 + dim, jnp.int32)])
                diff = si - sj
                d2e = d2e + diff * diff
            w_v[pl.ds(r * K, K)] = jnp.exp(-10.0 * d2e)
            idxsel_v[pl.ds(r * K, K)] = bi
            return 0

        lax.fori_loop(0, CHUNK, row_scan, 0)
        pltpu.async_copy(h_hbm.at[idxsel_v], h_v, sem).wait()

        def row_agg(r, _):
            def k_body(k, carry):
                acc, mx = carry
                hk = h_v[r * K + k, :]
                wk = plsc.load_gather(
                    w_v, [jnp.full((16,), r * K + k, jnp.int32)])
                msg = hk * wk
                return acc + msg, jnp.maximum(mx, msg)

            acc, mx = lax.fori_loop(
                0, K, k_body,
                (jnp.zeros((16,), jnp.float32),
                 jnp.full((16,), -BIGF, jnp.float32)))
            mean_f[pl.ds(r * K, K)] = acc * (1.0 / 16.0)
            max_f[pl.ds(r * K, K)] = mx
            return 0

        lax.fori_loop(0, CHUNK, row_agg, 0)
        pltpu.sync_copy(mean_f, mean_hbm.at[pl.ds(row0 * K, CHUNK * K)])
        pltpu.sync_copy(max_f, max_hbm.at[pl.ds(row0 * K, CHUNK * K)])
        pltpu.sync_copy(idxsel_v, idx_hbm.at[pl.ds(row0 * K, CHUNK * K)])
        pltpu.sync_copy(w_v, w_hbm.at[pl.ds(row0 * K, CHUNK * K)])
        return 0

    lax.fori_loop(0, NCHUNK, chunk_body, 0)


def _scagg(d2_flat, s4_flat, h):
    mesh = plsc.VectorSubcoreMesh(core_axis_name="c", subcore_axis_name="s")
    fn = functools.partial(
        pl.kernel, _scagg_body, mesh=mesh,
        compiler_params=pltpu.CompilerParams(needs_layout_passes=False,
                                             use_tc_tiling_on_sc=False),
        out_type=[
            jax.ShapeDtypeStruct((NP * K,), jnp.float32),
            jax.ShapeDtypeStruct((NP * K,), jnp.float32),
            jax.ShapeDtypeStruct((NP * K,), jnp.int32),
            jax.ShapeDtypeStruct((NP * K,), jnp.float32),
        ],
        scratch_types=[
            pltpu.VMEM((CHUNK * NP,), jnp.float32),
            pltpu.VMEM((NP * 4,), jnp.float32),
            pltpu.VMEM((CHUNK * K,), jnp.int32),
            pltpu.VMEM((CHUNK * K,), jnp.float32),
            pltpu.VMEM((CHUNK * K, 16), jnp.float32),
            pltpu.VMEM((CHUNK * K,), jnp.float32),
            pltpu.VMEM((CHUNK * K,), jnp.float32),
            pltpu.SemaphoreType.DMA,
        ],
    )()
    mean, mx, idx, w = fn(d2_flat, s4_flat, h)
    return mean.reshape(NP, K), mx.reshape(NP, K), idx.reshape(NP, K), w.reshape(NP, K)


def _outproj_body(d_ref, mean_ref, max_ref, wo, bo, out_ref):
    cat = jnp.concatenate([d_ref[...], mean_ref[...], max_ref[...]], axis=1)
    out_ref[...] = jnp.dot(cat, wo[...]) + bo[...]


def _outproj(d, mean_agg, max_agg, wo, bo):
    full = lambda shape: pl.BlockSpec(shape, lambda r: (0, 0))
    return pl.pallas_call(
        _outproj_body,
        grid=(NP // 256,),
        in_specs=[
            pl.BlockSpec((256, 64), lambda r: (r, 0)),
            pl.BlockSpec((256, 16), lambda r: (r, 0)),
            pl.BlockSpec((256, 16), lambda r: (r, 0)),
            full((96, 64)), full((1, 64)),
        ],
        out_specs=pl.BlockSpec((256, 64), lambda r: (r, 0)),
        out_shape=jax.ShapeDtypeStruct((NP, 64), jnp.float32),
    )(d, mean_agg, max_agg, wo, bo)


def _head_body(o0, o1, o2, o3, w1, b1, w2, b2, out_ref):
    ms = [jnp.max(o[...], axis=0, keepdims=True) for o in (o0, o1, o2, o3)]
    cat = jnp.concatenate(ms, axis=1)
    t = jnp.maximum(jnp.dot(cat, w1[...]) + b1[...], 0.0)
    out_ref[...] = jnp.dot(t, w2[...]) + b2[...]


def _head(outs, w1, b1, w2, b2):
    return pl.pallas_call(
        _head_body,
        out_shape=jax.ShapeDtypeStruct((1, 8), jnp.float32),
    )(*outs, w1, b1, w2, b2)


def kernel(x, b0_W1, b0_b1, b0_W2, b0_b2, b0_W3, b0_b3, b0_Ws, b0_bs, b0_Wh, b0_bh, b0_Wo, b0_bo, b1_W1, b1_b1, b1_W2, b1_b2, b1_W3, b1_b3, b1_Ws, b1_bs, b1_Wh, b1_bh, b1_Wo, b1_bo, b2_W1, b2_b1, b2_W2, b2_b2, b2_W3, b2_b3, b2_Ws, b2_bs, b2_Wh, b2_bh, b2_Wo, b2_bo, b3_W1, b3_b1, b3_W2, b3_b2, b3_W3, b3_b3, b3_Ws, b3_bs, b3_Wh, b3_bh, b3_Wo, b3_bo, d_W1, d_b1, d_W2, d_b2):
    blocks = [
        (b0_W1, b0_b1, b0_W2, b0_b2, b0_W3, b0_b3, b0_Ws, b0_bs, b0_Wh, b0_bh, b0_Wo, b0_bo),
        (b1_W1, b1_b1, b1_W2, b1_b2, b1_W3, b1_b3, b1_Ws, b1_bs, b1_Wh, b1_bh, b1_Wo, b1_bo),
        (b2_W1, b2_b1, b2_W2, b2_b2, b2_W3, b2_b3, b2_Ws, b2_bs, b2_Wh, b2_bh, b2_Wo, b2_bo),
        (b3_W1, b3_b1, b3_W2, b3_b2, b3_W3, b3_b3, b3_Ws, b3_bs, b3_Wh, b3_bh, b3_Wo, b3_bo),
    ]
    cur = jnp.pad(x[0], ((0, NP - N), (0, 0)))
    outs = []
    for (w1, b1, w2, b2, w3, b3, ws, bs, wh, bh, wo, bo) in blocks:
        d, s, h = _mlp(cur, w1, b1.reshape(1, -1), w2, b2.reshape(1, -1),
                       w3, b3.reshape(1, -1), ws, bs.reshape(1, -1),
                       wh, bh.reshape(1, -1))
        sq = jnp.sum(s * s, axis=1)
        sq = jnp.where(jnp.arange(NP) >= N, PADQ, sq)
        d2, _fold = _dist(s, sq.reshape(NP, 1), sq.reshape(1, NP))
        mean_agg, max_agg, _, _ = _scagg(d2.reshape(-1), s.reshape(-1), h)
        cur = _outproj(d, mean_agg, max_agg, wo, bo.reshape(1, -1))
        outs.append(cur[:N])
    res = _head(outs, d_W1, d_b1.reshape(1, -1), d_W2, d_b2.reshape(1, -1))
    return jnp.squeeze(res, axis=0)


# speedup vs baseline: 1.0024x; 1.0024x over previous
import functools

import jax
import jax.numpy as jnp
from jax import lax
from jax.experimental import pallas as pl
from jax.experimental.pallas import tpu as pltpu
from jax.experimental.pallas import tpu_sc as plsc

N = 10000
NP = 10240
K = 16
BIGF = 3.0e38
BIGI = 2 ** 30
PADQ = BIGF / 16
NSUB = 32
ROWS_PER_SUB = NP // NSUB
CHUNK = 8
NCHUNK = ROWS_PER_SUB // CHUNK
RB = 128


def _mlp_body(x_ref, w1, b1, w2, b2, w3, b3, ws, bs, wh, bh,
              d_ref, s_ref, h_ref):
    x = x_ref[...]
    t = jnp.tanh(jnp.dot(x, w1[...]) + b1[...])
    t = jnp.tanh(jnp.dot(t, w2[...]) + b2[...])
    d = jnp.dot(t, w3[...]) + b3[...]
    s = jnp.dot(d, ws[...]) + bs[...]
    s_ref[...] = s
    d_ref[...] = d
    h_ref[...] = jnp.dot(d, wh[...]) + bh[...]


def _mlp(x, w1, b1, w2, b2, w3, b3, ws, bs, wh, bh):
    in_dim = x.shape[1]
    full = lambda shape: pl.BlockSpec(shape, lambda r: (0, 0))
    return pl.pallas_call(
        _mlp_body,
        grid=(NP // 256,),
        in_specs=[
            pl.BlockSpec((256, in_dim), lambda r: (r, 0)),
            full((in_dim, 64)), full((1, 64)),
            full((64, 64)), full((1, 64)),
            full((64, 64)), full((1, 64)),
            full((64, 4)), full((1, 4)),
            full((64, 16)), full((1, 16)),
        ],
        out_specs=[
            pl.BlockSpec((256, 64), lambda r: (r, 0)),
            pl.BlockSpec((256, 4), lambda r: (r, 0)),
            pl.BlockSpec((256, 16), lambda r: (r, 0)),
        ],
        out_shape=[
            jax.ShapeDtypeStruct((NP, 64), jnp.float32),
            jax.ShapeDtypeStruct((NP, 4), jnp.float32),
            jax.ShapeDtypeStruct((NP, 16), jnp.float32),
        ],
    )(x, w1, b1, w2, b2, w3, b3, ws, bs, wh, bh)


def _dist_body(s_rows, s_all, sq_rows, sq_row_t, d2_ref):
    t = lax.dot_general(s_rows[...], s_all[...],
                        (((1,), (1,)), ((), ())))
    d2_ref[...] = (sq_rows[...] + sq_row_t[...]) - 2.0 * t


def _dist(s, sq, sq_t):
    return pl.pallas_call(
        _dist_body,
        grid=(NP // RB,),
        in_specs=[
            pl.BlockSpec((RB, 4), lambda r: (r, 0)),
            pl.BlockSpec((NP, 4), lambda r: (0, 0)),
            pl.BlockSpec((RB, 1), lambda r: (r, 0)),
            pl.BlockSpec((1, NP), lambda r: (0, 0)),
        ],
        out_specs=pl.BlockSpec((RB, NP), lambda r: (r, 0)),
        out_shape=jax.ShapeDtypeStruct((NP, NP), jnp.float32),
    )(s, s, sq, sq_t)


def _scagg_body(d2f_hbm, s4f_hbm, h_hbm, mean_hbm, max_hbm,
                buf, s4v, idxsel_v, w_v, h_v, mean_f, max_f, sem):
    cid = lax.axis_index("c")
    sid = lax.axis_index("s")
    wid = sid * 2 + cid
    base = wid * ROWS_PER_SUB
    pltpu.sync_copy(s4f_hbm, s4v)
    iota16 = lax.iota(jnp.int32, 16)

    def insert_group(v, vidx, st):
        bv, bi, tau, ei = st

        def w_cond(c):
            bv, bi, v, tau, ei = c
            return jnp.any((v < tau) | ((v == tau) & (vidx < ei)))

        def w_body(c):
            bv, bi, v, tau, ei = c
            mn = jnp.min(v)
            ci = jnp.min(jnp.where(v == mn, vidx, BIGI))
            qm = bi == ei
            bv = jnp.where(qm, mn, bv)
            bi = jnp.where(qm, ci, bi)
            v = jnp.where(vidx == ci, BIGF, v)
            tau = jnp.max(bv)
            ei = jnp.max(jnp.where(bv == tau, bi, -BIGI))
            return bv, bi, v, tau, ei

        bv, bi, _, tau, ei = lax.while_loop(
            w_cond, w_body, (bv, bi, v, tau, ei))
        return bv, bi, tau, ei

    def chunk_body(ci_, _):
        row0 = base + ci_ * CHUNK
        pltpu.sync_copy(d2f_hbm.at[pl.ds(row0 * NP, CHUNK * NP)], buf)

        def row_scan(r, _):
            row = row0 + r

            def g4_body(i, st):
                off = r * NP + i * 64
                v0 = buf[pl.ds(off, 16)]
                v1 = buf[pl.ds(off + 16, 16)]
                v2 = buf[pl.ds(off + 32, 16)]
                v3 = buf[pl.ds(off + 48, 16)]
                tau = st[2]
                hit = jnp.any((v0 <= tau) | (v1 <= tau)
                              | (v2 <= tau) | (v3 <= tau))

                def do_insert(st):
                    for j, v in enumerate((v0, v1, v2, v3)):
                        st = insert_group(v, iota16 + (i * 64 + j * 16), st)
                    return st

                return lax.cond(hit, do_insert, lambda st: st, st)

            st0 = (jnp.full((16,), BIGF, jnp.float32), iota16 - 16,
                   jnp.float32(BIGF), jnp.int32(-1))
            bv, bi, _, _ = lax.fori_loop(0, NP // 64, g4_body, st0)

            d2e = jnp.zeros((16,), jnp.float32)
            for dim in range(4):
                sj = plsc.load_gather(s4v, [bi * 4 + dim])
                si = plsc.load_gather(
                    s4v, [jnp.full((16,), row * 4 + dim, jnp.int32)])
                diff = si - sj
                d2e = d2e + diff * diff
            w_v[pl.ds(r * K, K)] = jnp.exp(-10.0 * d2e)
            idxsel_v[pl.ds(r * K, K)] = bi
            return 0

        lax.fori_loop(0, CHUNK, row_scan, 0)
        pltpu.async_copy(h_hbm.at[idxsel_v], h_v, sem).wait()

        def row_agg(r, _):
            def k_body(k, carry):
                acc, mx = carry
                hk = h_v[r * K + k, :]
                wk = plsc.load_gather(
                    w_v, [jnp.full((16,), r * K + k, jnp.int32)])
                msg = hk * wk
                return acc + msg, jnp.maximum(mx, msg)

            acc, mx = lax.fori_loop(
                0, K, k_body,
                (jnp.zeros((16,), jnp.float32),
                 jnp.full((16,), -BIGF, jnp.float32)))
            mean_f[pl.ds(r * K, K)] = acc * (1.0 / 16.0)
            max_f[pl.ds(r * K, K)] = mx
            return 0

        lax.fori_loop(0, CHUNK, row_agg, 0)
        pltpu.sync_copy(mean_f, mean_hbm.at[pl.ds(row0 * K, CHUNK * K)])
        pltpu.sync_copy(max_f, max_hbm.at[pl.ds(row0 * K, CHUNK * K)])
        return 0

    lax.fori_loop(0, NCHUNK, chunk_body, 0)


def _scagg(d2_flat, s4_flat, h):
    mesh = plsc.VectorSubcoreMesh(core_axis_name="c", subcore_axis_name="s")
    fn = functools.partial(
        pl.kernel, _scagg_body, mesh=mesh,
        compiler_params=pltpu.CompilerParams(needs_layout_passes=False,
                                             use_tc_tiling_on_sc=False),
        out_type=[
            jax.ShapeDtypeStruct((NP * K,), jnp.float32),
            jax.ShapeDtypeStruct((NP * K,), jnp.float32),
        ],
        scratch_types=[
            pltpu.VMEM((CHUNK * NP,), jnp.float32),
            pltpu.VMEM((NP * 4,), jnp.float32),
            pltpu.VMEM((CHUNK * K,), jnp.int32),
            pltpu.VMEM((CHUNK * K,), jnp.float32),
            pltpu.VMEM((CHUNK * K, 16), jnp.float32),
            pltpu.VMEM((CHUNK * K,), jnp.float32),
            pltpu.VMEM((CHUNK * K,), jnp.float32),
            pltpu.SemaphoreType.DMA,
        ],
    )()
    mean, mx = fn(d2_flat, s4_flat, h)
    return mean.reshape(NP, K), mx.reshape(NP, K)


def _outproj_body(d_ref, mean_ref, max_ref, wo, bo, out_ref):
    cat = jnp.concatenate([d_ref[...], mean_ref[...], max_ref[...]], axis=1)
    out_ref[...] = jnp.dot(cat, wo[...]) + bo[...]


def _outproj(d, mean_agg, max_agg, wo, bo):
    full = lambda shape: pl.BlockSpec(shape, lambda r: (0, 0))
    return pl.pallas_call(
        _outproj_body,
        grid=(NP // 256,),
        in_specs=[
            pl.BlockSpec((256, 64), lambda r: (r, 0)),
            pl.BlockSpec((256, 16), lambda r: (r, 0)),
            pl.BlockSpec((256, 16), lambda r: (r, 0)),
            full((96, 64)), full((1, 64)),
        ],
        out_specs=pl.BlockSpec((256, 64), lambda r: (r, 0)),
        out_shape=jax.ShapeDtypeStruct((NP, 64), jnp.float32),
    )(d, mean_agg, max_agg, wo, bo)


def _head_body(o0, o1, o2, o3, w1, b1, w2, b2, out_ref):
    ms = [jnp.max(o[...], axis=0, keepdims=True) for o in (o0, o1, o2, o3)]
    cat = jnp.concatenate(ms, axis=1)
    t = jnp.maximum(jnp.dot(cat, w1[...]) + b1[...], 0.0)
    out_ref[...] = jnp.dot(t, w2[...]) + b2[...]


def _head(outs, w1, b1, w2, b2):
    return pl.pallas_call(
        _head_body,
        out_shape=jax.ShapeDtypeStruct((1, 8), jnp.float32),
    )(*outs, w1, b1, w2, b2)


def kernel(x, b0_W1, b0_b1, b0_W2, b0_b2, b0_W3, b0_b3, b0_Ws, b0_bs, b0_Wh, b0_bh, b0_Wo, b0_bo, b1_W1, b1_b1, b1_W2, b1_b2, b1_W3, b1_b3, b1_Ws, b1_bs, b1_Wh, b1_bh, b1_Wo, b1_bo, b2_W1, b2_b1, b2_W2, b2_b2, b2_W3, b2_b3, b2_Ws, b2_bs, b2_Wh, b2_bh, b2_Wo, b2_bo, b3_W1, b3_b1, b3_W2, b3_b2, b3_W3, b3_b3, b3_Ws, b3_bs, b3_Wh, b3_bh, b3_Wo, b3_bo, d_W1, d_b1, d_W2, d_b2):
    blocks = [
        (b0_W1, b0_b1, b0_W2, b0_b2, b0_W3, b0_b3, b0_Ws, b0_bs, b0_Wh, b0_bh, b0_Wo, b0_bo),
        (b1_W1, b1_b1, b1_W2, b1_b2, b1_W3, b1_b3, b1_Ws, b1_bs, b1_Wh, b1_bh, b1_Wo, b1_bo),
        (b2_W1, b2_b1, b2_W2, b2_b2, b2_W3, b2_b3, b2_Ws, b2_bs, b2_Wh, b2_bh, b2_Wo, b2_bo),
        (b3_W1, b3_b1, b3_W2, b3_b2, b3_W3, b3_b3, b3_Ws, b3_bs, b3_Wh, b3_bh, b3_Wo, b3_bo),
    ]
    cur = jnp.pad(x[0], ((0, NP - N), (0, 0)))
    outs = []
    for (w1, b1, w2, b2, w3, b3, ws, bs, wh, bh, wo, bo) in blocks:
        d, s, h = _mlp(cur, w1, b1.reshape(1, -1), w2, b2.reshape(1, -1),
                       w3, b3.reshape(1, -1), ws, bs.reshape(1, -1),
                       wh, bh.reshape(1, -1))
        sq = jnp.sum(s * s, axis=1)
        sq = jnp.where(jnp.arange(NP) >= N, PADQ, sq)
        d2 = _dist(s, sq.reshape(NP, 1), sq.reshape(1, NP))
        mean_agg, max_agg = _scagg(d2.reshape(-1), s.reshape(-1), h)
        cur = _outproj(d, mean_agg, max_agg, wo, bo.reshape(1, -1))
        outs.append(cur[:N])
    res = _head(outs, d_W1, d_b1.reshape(1, -1), d_W2, d_b2.reshape(1, -1))
    return jnp.squeeze(res, axis=0)
